# baseline (speedup 1.0000x reference)
import functools

import jax
import jax.numpy as jnp
from jax.experimental import pallas as pl
import jax.experimental.pallas.tpu as pltpu

BR = 80
CB = 640


def _s1_kernel(x_ref, W1_ref, s1b_ref):
    s1 = jnp.dot(x_ref[...], W1_ref[...], preferred_element_type=jnp.float32)
    s1b_ref[...] = s1.astype(jnp.bfloat16)


def _main_kernel(adj_ref, s1b_ref, b1_ref, z0_ref, t_ref, ht_out_ref,
                 arena_ref, stage_ref, big_ref, *, n, nb, nh):
    i = pl.program_id(0)
    b0 = z0_ref[0]
    g = CB // BR
    nc = max(n // CB - 1, 1)
    npad = nc * CB
    starts = [CB * c * (c + 1) // 2 for c in range(nc)]
    last_cached = nc * g

    @pl.when(i == 0)
    def _():
        big_ref[:, :nh] = s1b_ref[...]
        big_ref[:, nh:] = jnp.zeros((n, 3 * nh), jnp.bfloat16)
        t_ref[...] = jnp.zeros((n, nh), jnp.float32)

    ab = adj_ref[...].astype(jnp.bfloat16)
    both = jnp.dot(ab, big_ref[:, :2 * nh],
                   preferred_element_type=jnp.float32)
    h_i = jnp.maximum(both[:, :nh] + b1_ref[...], 0.0)
    t_ref[pl.ds(i * BR, BR), :] += both[:, nh:]
    big_ref[pl.ds(i * BR, BR), 2 * nh:3 * nh] = h_i.astype(jnp.bfloat16)

    @pl.when(i < last_cached)
    def _():
        q = jnp.round(adj_ref[:, :nc * CB] * 15.0 - 8.0).astype(jnp.bfloat16)

        @pl.when(i % 2 == 0)
        def _():
            stage_ref[...] = q

        @pl.when(i % 2 == 1)
        def _():
            c_min = i // g
            pair_row = 160 * (i // 2)
            for c in range(nc):
                @pl.when(c >= c_min)
                def _(c=c):
                    pair = jnp.concatenate(
                        [stage_ref[:, pl.ds(c * CB + 128 * b0, CB)],
                         q[:, c * CB:(c + 1) * CB]], axis=0)
                    arena_ref[pl.ds(starts[c] + pair_row, 2 * BR), :] = (
                        pair.astype(jnp.int8))

    for c in range(nc):
        @pl.when(i == g * (c + 1) - 1)
        def _(c=c):
            h_chunk = big_ref[pl.ds(c * CB + 16 * b0, CB), 2 * nh:3 * nh]
            big_ref[c * CB:(c + 1) * CB, nh:2 * nh] = h_chunk
            qa = arena_ref[pl.ds(starts[c] + 32 * b0, npad),
                           :].astype(jnp.bfloat16)
            araw = jnp.dot(qa, h_chunk, preferred_element_type=jnp.float32)
            csum = jnp.sum(h_chunk.astype(jnp.float32), axis=0, keepdims=True)
            contrib = (araw + 8.0 * csum) / 15.0
            rows = jax.lax.broadcasted_iota(jnp.int32, (npad, nh), 0)
            t_ref[pl.ds(8 * b0, npad), :] += jnp.where(
                rows < CB * (c + 1), contrib, 0.0)

    @pl.when(i == nb - 1)
    def _():
        ht_out_ref[...] = big_ref[npad:n, 2 * nh:3 * nh]


def _epilogue_kernel(t_ref, adjt_ref, ht_ref, W2_ref, b2_ref, Wl_ref, bl_ref,
                     logp_ref, z_ref):
    t = t_ref[...] + jnp.dot(adjt_ref[...].astype(jnp.bfloat16), ht_ref[...],
                             preferred_element_type=jnp.float32)
    zb = jnp.dot(t, W2_ref[...],
                 preferred_element_type=jnp.float32) + b2_ref[...]
    z_ref[...] = zb
    logits = jnp.dot(jnp.maximum(zb, 0.0), Wl_ref[...],
                     preferred_element_type=jnp.float32) + bl_ref[...]
    m = jnp.max(logits, axis=1, keepdims=True)
    s = logits - m
    logp_ref[...] = s - jnp.log(jnp.sum(jnp.exp(s), axis=1, keepdims=True))


@jax.jit
def kernel(x, adj, W1, b1, W2, b2, Wl, bl):
    n, in_feat = x.shape
    nh = W1.shape[1]
    n_emb = W2.shape[1]
    n_class = Wl.shape[1]
    nb = n // BR
    nc = max(n // CB - 1, 1)
    npad = nc * CB
    tail = n - npad
    arena_rows = CB * nc * (nc + 1) // 2

    s1b = pl.pallas_call(
        _s1_kernel,
        grid=(1,),
        in_specs=[pl.BlockSpec((n, in_feat), lambda i: (0, 0)),
                  pl.BlockSpec((in_feat, nh), lambda i: (0, 0))],
        out_specs=pl.BlockSpec((n, nh), lambda i: (0, 0)),
        out_shape=jax.ShapeDtypeStruct((n, nh), jnp.bfloat16),
    )(x, W1)

    t, h_tail = pl.pallas_call(
        functools.partial(_main_kernel, n=n, nb=nb, nh=nh),
        grid=(nb,),
        in_specs=[
            pl.BlockSpec((BR, n), lambda i: (i, 0)),
            pl.BlockSpec((n, nh), lambda i: (0, 0)),
            pl.BlockSpec((1, nh), lambda i: (0, 0)),
            pl.BlockSpec(memory_space=pltpu.SMEM),
        ],
        out_specs=[
            pl.BlockSpec((n, nh), lambda i: (0, 0)),
            pl.BlockSpec((tail, nh), lambda i: (0, 0)),
        ],
        out_shape=[
            jax.ShapeDtypeStruct((n, nh), jnp.float32),
            jax.ShapeDtypeStruct((tail, nh), jnp.bfloat16),
        ],
        scratch_shapes=[
            pltpu.VMEM((arena_rows, CB), jnp.int8),
            pltpu.VMEM((BR, npad), jnp.bfloat16),
            pltpu.VMEM((n, 4 * nh), jnp.bfloat16),
        ],
        compiler_params=pltpu.CompilerParams(
            vmem_limit_bytes=64 * 1024 * 1024),
    )(adj, s1b, b1.reshape(1, nh), jnp.zeros((1,), jnp.int32))

    EB = n // 10
    adj_tail = adj[:, npad:]
    logp, z = pl.pallas_call(
        _epilogue_kernel,
        grid=(n // EB,),
        in_specs=[
            pl.BlockSpec((EB, nh), lambda i: (i, 0)),
            pl.BlockSpec((EB, tail), lambda i: (i, 0)),
            pl.BlockSpec((tail, nh), lambda i: (0, 0)),
            pl.BlockSpec((nh, n_emb), lambda i: (0, 0)),
            pl.BlockSpec((1, n_emb), lambda i: (0, 0)),
            pl.BlockSpec((n_emb, n_class), lambda i: (0, 0)),
            pl.BlockSpec((1, n_class), lambda i: (0, 0)),
        ],
        out_specs=[
            pl.BlockSpec((EB, n_class), lambda i: (i, 0)),
            pl.BlockSpec((EB, n_emb), lambda i: (i, 0)),
        ],
        out_shape=[
            jax.ShapeDtypeStruct((n, n_class), jnp.float32),
            jax.ShapeDtypeStruct((n, n_emb), jnp.float32),
        ],
    )(t, adj_tail, h_tail, W2, b2.reshape(1, n_emb), Wl,
      bl.reshape(1, n_class))
    return (logp, z)


# speedup vs baseline: 18.6909x; 18.6909x over previous
import functools

import jax
import jax.numpy as jnp
from jax.experimental import pallas as pl
import jax.experimental.pallas.tpu as pltpu

BR = 400


def _s1_kernel(x_ref, W1_ref, s1b_ref):
    s1 = jnp.dot(x_ref[...], W1_ref[...], preferred_element_type=jnp.float32)
    s1b_ref[...] = s1.astype(jnp.bfloat16)


def _pass1_kernel(adj_ref, s1b_ref, b1_ref, q8_ref, h_ref, *, nh):
    i = pl.program_id(0)
    ab = adj_ref[...].astype(jnp.bfloat16)
    h = jnp.maximum(
        jnp.dot(ab, s1b_ref[...], preferred_element_type=jnp.float32)
        + b1_ref[...], 0.0)
    h_ref[pl.ds(i * BR, BR), :] = h.astype(jnp.bfloat16)
    q8_ref[...] = adj_ref[...].astype(jnp.float8_e4m3fn)


def _pass2_kernel(q8_ref, h_ref, W2_ref, b2_ref, Wl_ref, bl_ref,
                  logp_ref, z_ref):
    h8 = (h_ref[...] * (1.0 / 16.0)).astype(jnp.float8_e4m3fn)
    t = jnp.dot(q8_ref[...], h8, preferred_element_type=jnp.float32) * 16.0
    zb = jnp.dot(t, W2_ref[...],
                 preferred_element_type=jnp.float32) + b2_ref[...]
    z_ref[...] = zb
    logits = jnp.dot(jnp.maximum(zb, 0.0), Wl_ref[...],
                     preferred_element_type=jnp.float32) + bl_ref[...]
    m = jnp.max(logits, axis=1, keepdims=True)
    s = logits - m
    logp_ref[...] = s - jnp.log(jnp.sum(jnp.exp(s), axis=1, keepdims=True))


@jax.jit
def kernel(x, adj, W1, b1, W2, b2, Wl, bl):
    n, in_feat = x.shape
    nh = W1.shape[1]
    n_emb = W2.shape[1]
    n_class = Wl.shape[1]
    nb = n // BR

    s1b = pl.pallas_call(
        _s1_kernel,
        grid=(1,),
        in_specs=[pl.BlockSpec((n, in_feat), lambda i: (0, 0)),
                  pl.BlockSpec((in_feat, nh), lambda i: (0, 0))],
        out_specs=pl.BlockSpec((n, nh), lambda i: (0, 0)),
        out_shape=jax.ShapeDtypeStruct((n, nh), jnp.bfloat16),
    )(x, W1)

    q8, h = pl.pallas_call(
        functools.partial(_pass1_kernel, nh=nh),
        grid=(nb,),
        in_specs=[
            pl.BlockSpec((BR, n), lambda i: (i, 0)),
            pl.BlockSpec((n, nh), lambda i: (0, 0)),
            pl.BlockSpec((1, nh), lambda i: (0, 0)),
        ],
        out_specs=[
            pl.BlockSpec((BR, n), lambda i: (i, 0)),
            pl.BlockSpec((n, nh), lambda i: (0, 0)),
        ],
        out_shape=[
            jax.ShapeDtypeStruct((n, n), jnp.float8_e4m3fn),
            jax.ShapeDtypeStruct((n, nh), jnp.bfloat16),
        ],
    )(adj, s1b, b1.reshape(1, nh))

    logp, z = pl.pallas_call(
        _pass2_kernel,
        grid=(nb,),
        in_specs=[
            pl.BlockSpec((BR, n), lambda i: (i, 0)),
            pl.BlockSpec((n, nh), lambda i: (0, 0)),
            pl.BlockSpec((nh, n_emb), lambda i: (0, 0)),
            pl.BlockSpec((1, n_emb), lambda i: (0, 0)),
            pl.BlockSpec((n_emb, n_class), lambda i: (0, 0)),
            pl.BlockSpec((1, n_class), lambda i: (0, 0)),
        ],
        out_specs=[
            pl.BlockSpec((BR, n_class), lambda i: (i, 0)),
            pl.BlockSpec((BR, n_emb), lambda i: (i, 0)),
        ],
        out_shape=[
            jax.ShapeDtypeStruct((n, n_class), jnp.float32),
            jax.ShapeDtypeStruct((n, n_emb), jnp.float32),
        ],
    )(q8, h, W2, b2.reshape(1, n_emb), Wl, bl.reshape(1, n_class))
    return (logp, z)


# speedup vs baseline: 18.8076x; 1.0062x over previous
import functools

import jax
import jax.numpy as jnp
from jax.experimental import pallas as pl
import jax.experimental.pallas.tpu as pltpu

BR = 400


def _s1_kernel(x_ref, W1_ref, s1b_ref):
    s1 = jnp.dot(x_ref[...], W1_ref[...], preferred_element_type=jnp.float32)
    s1b_ref[...] = s1.astype(jnp.bfloat16)


def _pass1_kernel(adj_ref, s1b_ref, b1_ref, q8_ref, h8_ref, *, nh):
    ab = adj_ref[...].astype(jnp.bfloat16)
    h = jnp.maximum(
        jnp.dot(ab, s1b_ref[...], preferred_element_type=jnp.float32)
        + b1_ref[...], 0.0)
    h8_ref[...] = (h * (1.0 / 16.0)).astype(jnp.float8_e4m3fn)
    q8_ref[...] = adj_ref[...].astype(jnp.float8_e4m3fn)


def _pass2_kernel(q8_ref, h8_ref, W2_ref, b2_ref, Wl_ref, bl_ref,
                  logp_ref, z_ref):
    t = jnp.dot(q8_ref[...], h8_ref[...],
                preferred_element_type=jnp.float32) * 16.0
    zb = jnp.dot(t, W2_ref[...],
                 preferred_element_type=jnp.float32) + b2_ref[...]
    z_ref[...] = zb
    logits = jnp.dot(jnp.maximum(zb, 0.0), Wl_ref[...],
                     preferred_element_type=jnp.float32) + bl_ref[...]
    m = jnp.max(logits, axis=1, keepdims=True)
    s = logits - m
    logp_ref[...] = s - jnp.log(jnp.sum(jnp.exp(s), axis=1, keepdims=True))


@jax.jit
def kernel(x, adj, W1, b1, W2, b2, Wl, bl):
    n, in_feat = x.shape
    nh = W1.shape[1]
    n_emb = W2.shape[1]
    n_class = Wl.shape[1]
    nb = n // BR

    s1b = pl.pallas_call(
        _s1_kernel,
        grid=(1,),
        in_specs=[pl.BlockSpec((n, in_feat), lambda i: (0, 0)),
                  pl.BlockSpec((in_feat, nh), lambda i: (0, 0))],
        out_specs=pl.BlockSpec((n, nh), lambda i: (0, 0)),
        out_shape=jax.ShapeDtypeStruct((n, nh), jnp.bfloat16),
    )(x, W1)

    q8, h = pl.pallas_call(
        functools.partial(_pass1_kernel, nh=nh),
        grid=(nb,),
        in_specs=[
            pl.BlockSpec((BR, n), lambda i: (i, 0)),
            pl.BlockSpec((n, nh), lambda i: (0, 0)),
            pl.BlockSpec((1, nh), lambda i: (0, 0)),
        ],
        out_specs=[
            pl.BlockSpec((BR, n), lambda i: (i, 0)),
            pl.BlockSpec((BR, nh), lambda i: (i, 0)),
        ],
        out_shape=[
            jax.ShapeDtypeStruct((n, n), jnp.float8_e4m3fn),
            jax.ShapeDtypeStruct((n, nh), jnp.float8_e4m3fn),
        ],
    )(adj, s1b, b1.reshape(1, nh))

    logp, z = pl.pallas_call(
        _pass2_kernel,
        grid=(nb,),
        in_specs=[
            pl.BlockSpec((BR, n), lambda i: (i, 0)),
            pl.BlockSpec((n, nh), lambda i: (0, 0)),
            pl.BlockSpec((nh, n_emb), lambda i: (0, 0)),
            pl.BlockSpec((1, n_emb), lambda i: (0, 0)),
            pl.BlockSpec((n_emb, n_class), lambda i: (0, 0)),
            pl.BlockSpec((1, n_class), lambda i: (0, 0)),
        ],
        out_specs=[
            pl.BlockSpec((BR, n_class), lambda i: (i, 0)),
            pl.BlockSpec((BR, n_emb), lambda i: (i, 0)),
        ],
        out_shape=[
            jax.ShapeDtypeStruct((n, n_class), jnp.float32),
            jax.ShapeDtypeStruct((n, n_emb), jnp.float32),
        ],
    )(q8, h, W2, b2.reshape(1, n_emb), Wl, bl.reshape(1, n_class))
    return (logp, z)


# speedup vs baseline: 18.8084x; 1.0000x over previous
import functools

import jax
import jax.numpy as jnp
from jax.experimental import pallas as pl

BR = 400


def _s1_kernel(x_ref, W1_ref, s1b_ref):
    s1 = jnp.dot(x_ref[...], W1_ref[...], preferred_element_type=jnp.float32)
    s1b_ref[...] = s1.astype(jnp.bfloat16)


def _pass1_kernel(adj_ref, s1b_ref, b1_ref, q8_ref, h8_ref, *, nh):
    ab = adj_ref[...].astype(jnp.bfloat16)
    h = jnp.maximum(
        jnp.dot(ab, s1b_ref[...], preferred_element_type=jnp.float32)
        + b1_ref[...], 0.0)
    h8_ref[...] = (h * (1.0 / 16.0)).astype(jnp.float8_e4m3fn)
    q8_ref[...] = adj_ref[...].astype(jnp.float8_e4m3fn)


def _pass2_kernel(q8_ref, h8_ref, W2_ref, b2_ref, Wl_ref, bl_ref,
                  logp_ref, z_ref):
    t = jnp.dot(q8_ref[...], h8_ref[...],
                preferred_element_type=jnp.float32) * 16.0
    zb = jnp.dot(t, W2_ref[...],
                 preferred_element_type=jnp.float32) + b2_ref[...]
    z_ref[...] = zb
    logits = jnp.dot(jnp.maximum(zb, 0.0), Wl_ref[...],
                     preferred_element_type=jnp.float32) + bl_ref[...]
    m = jnp.max(logits, axis=1, keepdims=True)
    s = logits - m
    logp_ref[...] = s - jnp.log(jnp.sum(jnp.exp(s), axis=1, keepdims=True))


@jax.jit
def kernel(x, adj, W1, b1, W2, b2, Wl, bl):
    n, in_feat = x.shape
    nh = W1.shape[1]
    n_emb = W2.shape[1]
    n_class = Wl.shape[1]
    nb = n // BR

    s1b = pl.pallas_call(
        _s1_kernel,
        grid=(1,),
        in_specs=[pl.BlockSpec((n, in_feat), lambda i: (0, 0)),
                  pl.BlockSpec((in_feat, nh), lambda i: (0, 0))],
        out_specs=pl.BlockSpec((n, nh), lambda i: (0, 0)),
        out_shape=jax.ShapeDtypeStruct((n, nh), jnp.bfloat16),
    )(x, W1)

    q8, h = pl.pallas_call(
        functools.partial(_pass1_kernel, nh=nh),
        grid=(nb,),
        in_specs=[
            pl.BlockSpec((BR, n), lambda i: (i, 0)),
            pl.BlockSpec((n, nh), lambda i: (0, 0)),
            pl.BlockSpec((1, nh), lambda i: (0, 0)),
        ],
        out_specs=[
            pl.BlockSpec((BR, n), lambda i: (i, 0)),
            pl.BlockSpec((BR, nh), lambda i: (i, 0)),
        ],
        out_shape=[
            jax.ShapeDtypeStruct((n, n), jnp.float8_e4m3fn),
            jax.ShapeDtypeStruct((n, nh), jnp.float8_e4m3fn),
        ],
    )(adj, s1b, b1.reshape(1, nh))

    logp, z = pl.pallas_call(
        _pass2_kernel,
        grid=(nb,),
        in_specs=[
            pl.BlockSpec((BR, n), lambda i: (i, 0)),
            pl.BlockSpec((n, nh), lambda i: (0, 0)),
            pl.BlockSpec((nh, n_emb), lambda i: (0, 0)),
            pl.BlockSpec((1, n_emb), lambda i: (0, 0)),
            pl.BlockSpec((n_emb, n_class), lambda i: (0, 0)),
            pl.BlockSpec((1, n_class), lambda i: (0, 0)),
        ],
        out_specs=[
            pl.BlockSpec((BR, n_class), lambda i: (i, 0)),
            pl.BlockSpec((BR, n_emb), lambda i: (i, 0)),
        ],
        out_shape=[
            jax.ShapeDtypeStruct((n, n_class), jnp.float32),
            jax.ShapeDtypeStruct((n, n_emb), jnp.float32),
        ],
    )(q8, h, W2, b2.reshape(1, n_emb), Wl, bl.reshape(1, n_class))
    return (logp, z)
